# baseline (speedup 1.0000x reference)
import jax
import jax.numpy as jnp
from jax import lax
from jax.experimental import pallas as pl
from jax.experimental.pallas import tpu as pltpu
from jax.experimental.pallas import tpu_sc as plsc

VOCAB = 1000000
BATCH = 4096
SEQ = 200
CH0 = 104
CH1 = SEQ - CH0
EMBED = 64
NUM_WORKERS = 32
ROWS_PER_W = BATCH // NUM_WORKERS
LANES = 16
D_CH = EMBED // LANES
PIPE = 4
N_CNT = SEQ // LANES + 1

VCHUNK = 128
N_VCHUNKS = VOCAB // VCHUNK
V_TAIL = VOCAB - N_VCHUNKS * VCHUNK
STEPS = (N_VCHUNKS + 2 * NUM_WORKERS - 1) // (2 * NUM_WORKERS)


def _transpose_body(wt_hbm, tail_hbm, out_hbm, buf_a, buf_b, flat_a, flat_b,
                    tail_buf, tail_flat, in_a, in_b, out_a, out_b):
    wid = lax.axis_index("s") * 2 + lax.axis_index("c")
    stride = lax.iota(jnp.int32, LANES) * EMBED

    def issue_in(cid, buf, sem):
        @pl.when(cid < N_VCHUNKS)
        def _():
            pltpu.async_copy(
                wt_hbm.at[pl.ds(0, EMBED), pl.ds(cid * VCHUNK, VCHUNK)],
                buf, sem)

    def transpose(buf, flat):
        def body(d, carry):
            for vb in range(VCHUNK // LANES):
                x = buf[d, pl.ds(vb * LANES, LANES)]
                plsc.store_scatter(
                    flat, [stride + (vb * LANES * EMBED + d)], x)
            return carry

        lax.fori_loop(0, EMBED, body, 0, unroll=4)

    def substep(j, cid, buf, flat, in_sem, out_sem):
        @pl.when(cid < N_VCHUNKS)
        def _():
            pltpu.make_async_copy(
                wt_hbm.at[pl.ds(0, EMBED), pl.ds(0, VCHUNK)], buf,
                in_sem).wait()

            @pl.when(j > 0)
            def _():
                pltpu.make_async_copy(
                    flat, out_hbm.at[pl.ds(0, VCHUNK * EMBED)],
                    out_sem).wait()

            transpose(buf, flat)
            pltpu.async_copy(
                flat, out_hbm.at[pl.ds(cid * VCHUNK * EMBED, VCHUNK * EMBED)],
                out_sem)
            issue_in(cid + 2 * NUM_WORKERS, buf, in_sem)

    issue_in(wid, buf_a, in_a)
    issue_in(NUM_WORKERS + wid, buf_b, in_b)

    def step(j, carry):
        substep(j, (2 * j) * NUM_WORKERS + wid, buf_a, flat_a, in_a, out_a)
        substep(j, (2 * j + 1) * NUM_WORKERS + wid, buf_b, flat_b, in_b, out_b)
        return carry

    lax.fori_loop(0, STEPS, step, 0)
    pltpu.make_async_copy(flat_a, out_hbm.at[pl.ds(0, VCHUNK * EMBED)],
                          out_a).wait()
    pltpu.make_async_copy(flat_b, out_hbm.at[pl.ds(0, VCHUNK * EMBED)],
                          out_b).wait()

    @pl.when(wid == 0)
    def _():
        pltpu.sync_copy(tail_hbm, tail_buf)

        def body(v, carry):
            for d in range(D_CH):
                tail_flat[pl.ds(v * EMBED + d * LANES, LANES)] = (
                    tail_buf[v, pl.ds(d * LANES, LANES)])
            return carry

        lax.fori_loop(0, V_TAIL, body, 0, unroll=4)
        pltpu.sync_copy(
            tail_flat,
            out_hbm.at[pl.ds(N_VCHUNKS * VCHUNK * EMBED, V_TAIL * EMBED)])


def _gather_body(idx_hbm, w_hbm, out_hbm, idx_v, rows_v, out_v, *sems):
    wid = lax.axis_index("s") * 2 + lax.axis_index("c")
    base = wid * ROWS_PER_W
    pltpu.sync_copy(idx_hbm.at[pl.ds(base, ROWS_PER_W)], idx_v)

    def issue(row, j):
        pltpu.async_copy(w_hbm.at[idx_v.at[row, pl.ds(0, CH0)]],
                         rows_v.at[j, pl.ds(0, CH0)], sems[j])
        pltpu.async_copy(w_hbm.at[idx_v.at[row, pl.ds(CH0, CH1)]],
                         rows_v.at[j, pl.ds(CH0, CH1)], sems[j])

    def drain(j):
        pltpu.make_async_copy(w_hbm.at[pl.ds(0, SEQ)],
                              rows_v.at[j], sems[j]).wait()

    lane = lax.iota(jnp.int32, LANES)

    def accumulate(row, j):
        cnt = jnp.zeros((LANES,), jnp.int32)
        for c in range(N_CNT):
            off = min(c * LANES, SEQ - LANES)
            iv = idx_v[row, pl.ds(off, LANES)]
            nz = iv != 0
            if c * LANES > off:
                nz = jnp.logical_and(nz, lane >= (c * LANES - off))
            cnt = cnt + plsc.all_reduce_population_count(nz)
        inv = 1.0 / jnp.maximum(cnt.astype(jnp.float32), 1.0)

        def inner(i, accs):
            return tuple(accs[d] + rows_v[j, i, pl.ds(d * LANES, LANES)]
                         for d in range(D_CH))

        zeros = tuple(jnp.zeros((LANES,), jnp.float32) for _ in range(D_CH))
        accs = lax.fori_loop(0, SEQ, inner, zeros, unroll=8)
        for d in range(D_CH):
            out_v[row, pl.ds(d * LANES, LANES)] = accs[d] * inv

    for j in range(PIPE):
        issue(j, j)

    def block_body(k, carry):
        for j in range(PIPE):
            row = k * PIPE + j
            drain(j)
            accumulate(row, j)

            @pl.when(row + PIPE < ROWS_PER_W)
            def _():
                issue(row + PIPE, j)
        return carry

    lax.fori_loop(0, ROWS_PER_W // PIPE, block_body, 0)
    pltpu.sync_copy(out_v, out_hbm.at[pl.ds(base, ROWS_PER_W)])


def kernel(sentence, weight):
    idx = sentence.astype(jnp.int32)

    transpose = pl.kernel(
        _transpose_body,
        out_type=jax.ShapeDtypeStruct((VOCAB * EMBED,), jnp.float32),
        mesh=plsc.VectorSubcoreMesh(core_axis_name="c", subcore_axis_name="s"),
        scratch_types=[
            pltpu.VMEM((EMBED, VCHUNK), jnp.float32),
            pltpu.VMEM((EMBED, VCHUNK), jnp.float32),
            pltpu.VMEM((VCHUNK * EMBED,), jnp.float32),
            pltpu.VMEM((VCHUNK * EMBED,), jnp.float32),
            pltpu.VMEM((V_TAIL, EMBED), jnp.float32),
            pltpu.VMEM((V_TAIL * EMBED,), jnp.float32),
            pltpu.SemaphoreType.DMA,
            pltpu.SemaphoreType.DMA,
            pltpu.SemaphoreType.DMA,
            pltpu.SemaphoreType.DMA,
        ],
        compiler_params=pltpu.CompilerParams(use_tc_tiling_on_sc=True,
                                             needs_layout_passes=False),
    )
    w_lin = transpose(weight.T, weight[N_VCHUNKS * VCHUNK:])
    w_lin = w_lin.reshape(VOCAB, EMBED)

    gather = pl.kernel(
        _gather_body,
        out_type=jax.ShapeDtypeStruct((BATCH, EMBED), jnp.float32),
        mesh=plsc.VectorSubcoreMesh(core_axis_name="c", subcore_axis_name="s"),
        scratch_types=[
            pltpu.VMEM((ROWS_PER_W, SEQ), jnp.int32),
            pltpu.VMEM((PIPE, SEQ, EMBED), jnp.float32),
            pltpu.VMEM((ROWS_PER_W, EMBED), jnp.float32),
        ] + [pltpu.SemaphoreType.DMA] * PIPE,
        compiler_params=pltpu.CompilerParams(use_tc_tiling_on_sc=False,
                                             needs_layout_passes=False),
    )
    return gather(idx, w_lin)


# speedup vs baseline: 1.9468x; 1.9468x over previous
import jax
import jax.numpy as jnp
from jax import lax
from jax.experimental import pallas as pl
from jax.experimental.pallas import tpu as pltpu
from jax.experimental.pallas import tpu_sc as plsc

VOCAB = 1000000
BATCH = 4096
SEQ = 200
CH0 = 104
CH1 = SEQ - CH0
EMBED = 64
NUM_WORKERS = 32
ROWS_PER_W = BATCH // NUM_WORKERS
LANES = 16
D_CH = EMBED // LANES
PIPE = 4
N_CNT = SEQ // LANES + 1

VCHUNK = 128
N_VCHUNKS = VOCAB // VCHUNK
V_TAIL = VOCAB - N_VCHUNKS * VCHUNK
STEPS = (N_VCHUNKS + 2 * NUM_WORKERS - 1) // (2 * NUM_WORKERS)


def _transpose_body(wt_hbm, tail_hbm, out_hbm, buf_a, buf_b, flat_a, flat_b,
                    tail_buf, tail_flat, in_a, in_b, out_a, out_b):
    wid = lax.axis_index("s") * 2 + lax.axis_index("c")
    iota = lax.iota(jnp.int32, LANES)
    d_base = [iota + db * LANES for db in range(EMBED // LANES)]

    def issue_in(cid, buf, sem):
        @pl.when(cid < N_VCHUNKS)
        def _():
            pltpu.async_copy(
                wt_hbm.at[pl.ds(0, EMBED), pl.ds(cid * VCHUNK, VCHUNK)],
                buf, sem)

    def transpose(buf, flat):
        for k in range(LANES):
            rot = jnp.bitwise_and(iota + k, LANES - 1)
            wk = rot * EMBED + iota

            def body(vb, carry, rot=rot, wk=wk):
                ridx = rot + vb * LANES
                wbase = wk + vb * (LANES * EMBED)
                for db in range(EMBED // LANES):
                    x = plsc.load_gather(buf, [d_base[db], ridx])
                    plsc.store_scatter(flat, [wbase + db * LANES], x)
                return carry

            lax.fori_loop(0, VCHUNK // LANES, body, 0, unroll=2)

    def substep(j, cid, buf, flat, in_sem, out_sem):
        @pl.when(cid < N_VCHUNKS)
        def _():
            pltpu.make_async_copy(
                wt_hbm.at[pl.ds(0, EMBED), pl.ds(0, VCHUNK)], buf,
                in_sem).wait()

            @pl.when(j > 0)
            def _():
                pltpu.make_async_copy(
                    flat, out_hbm.at[pl.ds(0, VCHUNK * EMBED)],
                    out_sem).wait()

            transpose(buf, flat)
            pltpu.async_copy(
                flat, out_hbm.at[pl.ds(cid * VCHUNK * EMBED, VCHUNK * EMBED)],
                out_sem)
            issue_in(cid + 2 * NUM_WORKERS, buf, in_sem)

    issue_in(wid, buf_a, in_a)
    issue_in(NUM_WORKERS + wid, buf_b, in_b)

    def step(j, carry):
        substep(j, (2 * j) * NUM_WORKERS + wid, buf_a, flat_a, in_a, out_a)
        substep(j, (2 * j + 1) * NUM_WORKERS + wid, buf_b, flat_b, in_b, out_b)
        return carry

    lax.fori_loop(0, STEPS, step, 0)
    pltpu.make_async_copy(flat_a, out_hbm.at[pl.ds(0, VCHUNK * EMBED)],
                          out_a).wait()
    pltpu.make_async_copy(flat_b, out_hbm.at[pl.ds(0, VCHUNK * EMBED)],
                          out_b).wait()

    @pl.when(wid == 0)
    def _():
        pltpu.sync_copy(tail_hbm, tail_buf)

        def body(v, carry):
            for d in range(D_CH):
                tail_flat[pl.ds(v * EMBED + d * LANES, LANES)] = (
                    tail_buf[v, pl.ds(d * LANES, LANES)])
            return carry

        lax.fori_loop(0, V_TAIL, body, 0, unroll=4)
        pltpu.sync_copy(
            tail_flat,
            out_hbm.at[pl.ds(N_VCHUNKS * VCHUNK * EMBED, V_TAIL * EMBED)])


def _gather_body(idx_hbm, w_hbm, out_hbm, idx_v, rows_v, out_v, *sems):
    wid = lax.axis_index("s") * 2 + lax.axis_index("c")
    base = wid * ROWS_PER_W
    pltpu.sync_copy(idx_hbm.at[pl.ds(base, ROWS_PER_W)], idx_v)

    def issue(row, j):
        pltpu.async_copy(w_hbm.at[idx_v.at[row, pl.ds(0, CH0)]],
                         rows_v.at[j, pl.ds(0, CH0)], sems[j])
        pltpu.async_copy(w_hbm.at[idx_v.at[row, pl.ds(CH0, CH1)]],
                         rows_v.at[j, pl.ds(CH0, CH1)], sems[j])

    def drain(j):
        pltpu.make_async_copy(w_hbm.at[pl.ds(0, SEQ)],
                              rows_v.at[j], sems[j]).wait()

    lane = lax.iota(jnp.int32, LANES)

    def accumulate(row, j):
        cnt = jnp.zeros((LANES,), jnp.int32)
        for c in range(N_CNT):
            off = min(c * LANES, SEQ - LANES)
            iv = idx_v[row, pl.ds(off, LANES)]
            nz = iv != 0
            if c * LANES > off:
                nz = jnp.logical_and(nz, lane >= (c * LANES - off))
            cnt = cnt + plsc.all_reduce_population_count(nz)
        inv = 1.0 / jnp.maximum(cnt.astype(jnp.float32), 1.0)

        def inner(i, accs):
            return tuple(accs[d] + rows_v[j, i, pl.ds(d * LANES, LANES)]
                         for d in range(D_CH))

        zeros = tuple(jnp.zeros((LANES,), jnp.float32) for _ in range(D_CH))
        accs = lax.fori_loop(0, SEQ, inner, zeros, unroll=8)
        for d in range(D_CH):
            out_v[row, pl.ds(d * LANES, LANES)] = accs[d] * inv

    for j in range(PIPE):
        issue(j, j)

    def block_body(k, carry):
        for j in range(PIPE):
            row = k * PIPE + j
            drain(j)
            accumulate(row, j)

            @pl.when(row + PIPE < ROWS_PER_W)
            def _():
                issue(row + PIPE, j)
        return carry

    lax.fori_loop(0, ROWS_PER_W // PIPE, block_body, 0)
    pltpu.sync_copy(out_v, out_hbm.at[pl.ds(base, ROWS_PER_W)])


def kernel(sentence, weight):
    idx = sentence.astype(jnp.int32)

    transpose = pl.kernel(
        _transpose_body,
        out_type=jax.ShapeDtypeStruct((VOCAB * EMBED,), jnp.float32),
        mesh=plsc.VectorSubcoreMesh(core_axis_name="c", subcore_axis_name="s"),
        scratch_types=[
            pltpu.VMEM((EMBED, VCHUNK), jnp.float32),
            pltpu.VMEM((EMBED, VCHUNK), jnp.float32),
            pltpu.VMEM((VCHUNK * EMBED,), jnp.float32),
            pltpu.VMEM((VCHUNK * EMBED,), jnp.float32),
            pltpu.VMEM((V_TAIL, EMBED), jnp.float32),
            pltpu.VMEM((V_TAIL * EMBED,), jnp.float32),
            pltpu.SemaphoreType.DMA,
            pltpu.SemaphoreType.DMA,
            pltpu.SemaphoreType.DMA,
            pltpu.SemaphoreType.DMA,
        ],
        compiler_params=pltpu.CompilerParams(use_tc_tiling_on_sc=True,
                                             needs_layout_passes=False),
    )
    w_lin = transpose(weight.T, weight[N_VCHUNKS * VCHUNK:])
    w_lin = w_lin.reshape(VOCAB, EMBED)

    gather = pl.kernel(
        _gather_body,
        out_type=jax.ShapeDtypeStruct((BATCH, EMBED), jnp.float32),
        mesh=plsc.VectorSubcoreMesh(core_axis_name="c", subcore_axis_name="s"),
        scratch_types=[
            pltpu.VMEM((ROWS_PER_W, SEQ), jnp.int32),
            pltpu.VMEM((PIPE, SEQ, EMBED), jnp.float32),
            pltpu.VMEM((ROWS_PER_W, EMBED), jnp.float32),
        ] + [pltpu.SemaphoreType.DMA] * PIPE,
        compiler_params=pltpu.CompilerParams(use_tc_tiling_on_sc=False,
                                             needs_layout_passes=False),
    )
    return gather(idx, w_lin)


# speedup vs baseline: 1.9825x; 1.0184x over previous
import jax
import jax.numpy as jnp
from jax import lax
from jax.experimental import pallas as pl
from jax.experimental.pallas import tpu as pltpu
from jax.experimental.pallas import tpu_sc as plsc

VOCAB = 1000000
BATCH = 4096
SEQ = 200
CH0 = 104
CH1 = SEQ - CH0
EMBED = 64
NUM_WORKERS = 32
ROWS_PER_W = BATCH // NUM_WORKERS
LANES = 16
D_CH = EMBED // LANES
PIPE = 4
N_CNT = SEQ // LANES + 1

VCHUNK = 128
N_VCHUNKS = VOCAB // VCHUNK
V_TAIL = VOCAB - N_VCHUNKS * VCHUNK
STEPS = (N_VCHUNKS + 2 * NUM_WORKERS - 1) // (2 * NUM_WORKERS)


def _transpose_body(wt_hbm, tail_hbm, out_hbm, buf_a, buf_b, flat_a, flat_b,
                    tail_buf, tail_flat, in_a, in_b, out_a, out_b):
    wid = lax.axis_index("s") * 2 + lax.axis_index("c")
    iota = lax.iota(jnp.int32, LANES)
    d_base = [iota + db * LANES for db in range(EMBED // LANES)]

    def issue_in(cid, buf, sem):
        @pl.when(cid < N_VCHUNKS)
        def _():
            pltpu.async_copy(
                wt_hbm.at[pl.ds(0, EMBED), pl.ds(cid * VCHUNK, VCHUNK)],
                buf, sem)

    def transpose(buf, flat):
        for k in range(LANES):
            rot = jnp.bitwise_and(iota + k, LANES - 1)
            wk = rot * EMBED + iota

            def body(vb, carry, rot=rot, wk=wk):
                ridx = rot + vb * LANES
                wbase = wk + vb * (LANES * EMBED)
                for db in range(EMBED // LANES):
                    x = plsc.load_gather(buf, [d_base[db], ridx])
                    plsc.store_scatter(flat, [wbase + db * LANES], x)
                return carry

            lax.fori_loop(0, VCHUNK // LANES, body, 0, unroll=4)

    def substep(j, cid, buf, flat, in_sem, out_sem):
        @pl.when(cid < N_VCHUNKS)
        def _():
            pltpu.make_async_copy(
                wt_hbm.at[pl.ds(0, EMBED), pl.ds(0, VCHUNK)], buf,
                in_sem).wait()

            @pl.when(j > 0)
            def _():
                pltpu.make_async_copy(
                    flat, out_hbm.at[pl.ds(0, VCHUNK * EMBED)],
                    out_sem).wait()

            transpose(buf, flat)
            pltpu.async_copy(
                flat, out_hbm.at[pl.ds(cid * VCHUNK * EMBED, VCHUNK * EMBED)],
                out_sem)
            issue_in(cid + 2 * NUM_WORKERS, buf, in_sem)

    issue_in(wid, buf_a, in_a)
    issue_in(NUM_WORKERS + wid, buf_b, in_b)

    def step(j, carry):
        substep(j, (2 * j) * NUM_WORKERS + wid, buf_a, flat_a, in_a, out_a)
        substep(j, (2 * j + 1) * NUM_WORKERS + wid, buf_b, flat_b, in_b, out_b)
        return carry

    lax.fori_loop(0, STEPS, step, 0)
    pltpu.make_async_copy(flat_a, out_hbm.at[pl.ds(0, VCHUNK * EMBED)],
                          out_a).wait()
    pltpu.make_async_copy(flat_b, out_hbm.at[pl.ds(0, VCHUNK * EMBED)],
                          out_b).wait()

    @pl.when(wid == 0)
    def _():
        pltpu.sync_copy(tail_hbm, tail_buf)

        def body(v, carry):
            for d in range(D_CH):
                tail_flat[pl.ds(v * EMBED + d * LANES, LANES)] = (
                    tail_buf[v, pl.ds(d * LANES, LANES)])
            return carry

        lax.fori_loop(0, V_TAIL, body, 0, unroll=4)
        pltpu.sync_copy(
            tail_flat,
            out_hbm.at[pl.ds(N_VCHUNKS * VCHUNK * EMBED, V_TAIL * EMBED)])


def _gather_body(idx_hbm, w_hbm, out_hbm, idx_v, rows_v, out_v, *sems):
    wid = lax.axis_index("s") * 2 + lax.axis_index("c")
    base = wid * ROWS_PER_W
    pltpu.sync_copy(idx_hbm.at[pl.ds(base, ROWS_PER_W)], idx_v)

    def issue(row, j):
        pltpu.async_copy(w_hbm.at[idx_v.at[row, pl.ds(0, CH0)]],
                         rows_v.at[j, pl.ds(0, CH0)], sems[j])
        pltpu.async_copy(w_hbm.at[idx_v.at[row, pl.ds(CH0, CH1)]],
                         rows_v.at[j, pl.ds(CH0, CH1)], sems[j])

    def drain(j):
        pltpu.make_async_copy(w_hbm.at[pl.ds(0, SEQ)],
                              rows_v.at[j], sems[j]).wait()

    lane = lax.iota(jnp.int32, LANES)

    def accumulate(row, j):
        cnt = jnp.zeros((LANES,), jnp.int32)
        for c in range(N_CNT):
            off = min(c * LANES, SEQ - LANES)
            iv = idx_v[row, pl.ds(off, LANES)]
            nz = iv != 0
            if c * LANES > off:
                nz = jnp.logical_and(nz, lane >= (c * LANES - off))
            cnt = cnt + plsc.all_reduce_population_count(nz)
        inv = 1.0 / jnp.maximum(cnt.astype(jnp.float32), 1.0)

        def inner(i, accs):
            return tuple(accs[d] + rows_v[j, i, pl.ds(d * LANES, LANES)]
                         for d in range(D_CH))

        zeros = tuple(jnp.zeros((LANES,), jnp.float32) for _ in range(D_CH))
        accs = lax.fori_loop(0, SEQ, inner, zeros, unroll=8)
        for d in range(D_CH):
            out_v[row, pl.ds(d * LANES, LANES)] = accs[d] * inv

    for j in range(PIPE):
        issue(j, j)

    def block_body(k, carry):
        for j in range(PIPE):
            row = k * PIPE + j
            drain(j)
            accumulate(row, j)

            @pl.when(row + PIPE < ROWS_PER_W)
            def _():
                issue(row + PIPE, j)
        return carry

    lax.fori_loop(0, ROWS_PER_W // PIPE, block_body, 0)
    pltpu.sync_copy(out_v, out_hbm.at[pl.ds(base, ROWS_PER_W)])


def kernel(sentence, weight):
    idx = sentence.astype(jnp.int32)

    transpose = pl.kernel(
        _transpose_body,
        out_type=jax.ShapeDtypeStruct((VOCAB * EMBED,), jnp.float32),
        mesh=plsc.VectorSubcoreMesh(core_axis_name="c", subcore_axis_name="s"),
        scratch_types=[
            pltpu.VMEM((EMBED, VCHUNK), jnp.float32),
            pltpu.VMEM((EMBED, VCHUNK), jnp.float32),
            pltpu.VMEM((VCHUNK * EMBED,), jnp.float32),
            pltpu.VMEM((VCHUNK * EMBED,), jnp.float32),
            pltpu.VMEM((V_TAIL, EMBED), jnp.float32),
            pltpu.VMEM((V_TAIL * EMBED,), jnp.float32),
            pltpu.SemaphoreType.DMA,
            pltpu.SemaphoreType.DMA,
            pltpu.SemaphoreType.DMA,
            pltpu.SemaphoreType.DMA,
        ],
        compiler_params=pltpu.CompilerParams(use_tc_tiling_on_sc=True,
                                             needs_layout_passes=False),
    )
    w_lin = transpose(weight.T, weight[N_VCHUNKS * VCHUNK:])
    w_lin = w_lin.reshape(VOCAB, EMBED)

    gather = pl.kernel(
        _gather_body,
        out_type=jax.ShapeDtypeStruct((BATCH, EMBED), jnp.float32),
        mesh=plsc.VectorSubcoreMesh(core_axis_name="c", subcore_axis_name="s"),
        scratch_types=[
            pltpu.VMEM((ROWS_PER_W, SEQ), jnp.int32),
            pltpu.VMEM((PIPE, SEQ, EMBED), jnp.float32),
            pltpu.VMEM((ROWS_PER_W, EMBED), jnp.float32),
        ] + [pltpu.SemaphoreType.DMA] * PIPE,
        compiler_params=pltpu.CompilerParams(use_tc_tiling_on_sc=False,
                                             needs_layout_passes=False),
    )
    return gather(idx, w_lin)


# speedup vs baseline: 2.2020x; 1.1107x over previous
import jax
import jax.numpy as jnp
from jax import lax
from jax.experimental import pallas as pl
from jax.experimental.pallas import tpu as pltpu
from jax.experimental.pallas import tpu_sc as plsc

VOCAB = 1000000
BATCH = 4096
SEQ = 200
CH0 = 104
CH1 = SEQ - CH0
EMBED = 64
NUM_WORKERS = 32
ROWS_PER_W = BATCH // NUM_WORKERS
LANES = 16
D_CH = EMBED // LANES
PIPE = 4
N_CNT = SEQ // LANES + 1

VCHUNK = 128
N_VCHUNKS = VOCAB // VCHUNK
V_TAIL = VOCAB - N_VCHUNKS * VCHUNK
STEPS = (N_VCHUNKS + 2 * NUM_WORKERS - 1) // (2 * NUM_WORKERS)


def _transpose_body(wt_hbm, tail_hbm, out_hbm, buf_a, buf_b, flat_a, flat_b,
                    tail_buf, tail_flat, in_a, in_b, out_a, out_b):
    wid = lax.axis_index("s") * 2 + lax.axis_index("c")
    iota = lax.iota(jnp.int32, LANES)
    d_base = [iota + db * LANES for db in range(EMBED // LANES)]

    def issue_in(cid, buf, sem):
        @pl.when(cid < N_VCHUNKS)
        def _():
            pltpu.async_copy(
                wt_hbm.at[pl.ds(0, EMBED), pl.ds(cid * VCHUNK, VCHUNK)],
                buf, sem)

    def transpose(buf, flat):
        def krot(k, carry):
            rot = jnp.bitwise_and(iota + k, LANES - 1)
            wk = rot * EMBED + iota
            for vb in range(VCHUNK // LANES):
                ridx = rot + vb * LANES
                wbase = wk + vb * (LANES * EMBED)
                for db in range(EMBED // LANES):
                    x = plsc.load_gather(buf, [d_base[db], ridx])
                    plsc.store_scatter(flat, [wbase + db * LANES], x)
            return carry

        lax.fori_loop(0, LANES, krot, 0, unroll=2)

    def substep(j, cid, buf, flat, in_sem, out_sem):
        @pl.when(cid < N_VCHUNKS)
        def _():
            pltpu.make_async_copy(
                wt_hbm.at[pl.ds(0, EMBED), pl.ds(0, VCHUNK)], buf,
                in_sem).wait()

            @pl.when(j > 0)
            def _():
                pltpu.make_async_copy(
                    flat, out_hbm.at[pl.ds(0, VCHUNK * EMBED)],
                    out_sem).wait()

            transpose(buf, flat)
            pltpu.async_copy(
                flat, out_hbm.at[pl.ds(cid * VCHUNK * EMBED, VCHUNK * EMBED)],
                out_sem)
            issue_in(cid + 2 * NUM_WORKERS, buf, in_sem)

    issue_in(wid, buf_a, in_a)
    issue_in(NUM_WORKERS + wid, buf_b, in_b)

    def step(j, carry):
        substep(j, (2 * j) * NUM_WORKERS + wid, buf_a, flat_a, in_a, out_a)
        substep(j, (2 * j + 1) * NUM_WORKERS + wid, buf_b, flat_b, in_b, out_b)
        return carry

    lax.fori_loop(0, STEPS, step, 0)
    pltpu.make_async_copy(flat_a, out_hbm.at[pl.ds(0, VCHUNK * EMBED)],
                          out_a).wait()
    pltpu.make_async_copy(flat_b, out_hbm.at[pl.ds(0, VCHUNK * EMBED)],
                          out_b).wait()

    @pl.when(wid == 0)
    def _():
        pltpu.sync_copy(tail_hbm, tail_buf)

        def body(v, carry):
            for d in range(D_CH):
                tail_flat[pl.ds(v * EMBED + d * LANES, LANES)] = (
                    tail_buf[v, pl.ds(d * LANES, LANES)])
            return carry

        lax.fori_loop(0, V_TAIL, body, 0, unroll=4)
        pltpu.sync_copy(
            tail_flat,
            out_hbm.at[pl.ds(N_VCHUNKS * VCHUNK * EMBED, V_TAIL * EMBED)])


def _gather_body(idx_hbm, w_hbm, out_hbm, idx_v, rows_v, out_v, *sems):
    wid = lax.axis_index("s") * 2 + lax.axis_index("c")
    base = wid * ROWS_PER_W
    pltpu.sync_copy(idx_hbm.at[pl.ds(base, ROWS_PER_W)], idx_v)

    def issue(row, j):
        pltpu.async_copy(w_hbm.at[idx_v.at[row, pl.ds(0, CH0)]],
                         rows_v.at[j, pl.ds(0, CH0)], sems[j])
        pltpu.async_copy(w_hbm.at[idx_v.at[row, pl.ds(CH0, CH1)]],
                         rows_v.at[j, pl.ds(CH0, CH1)], sems[j])

    def drain(j):
        pltpu.make_async_copy(w_hbm.at[pl.ds(0, SEQ)],
                              rows_v.at[j], sems[j]).wait()

    lane = lax.iota(jnp.int32, LANES)

    def accumulate(row, j):
        cnt = jnp.zeros((LANES,), jnp.int32)
        for c in range(N_CNT):
            off = min(c * LANES, SEQ - LANES)
            iv = idx_v[row, pl.ds(off, LANES)]
            nz = iv != 0
            if c * LANES > off:
                nz = jnp.logical_and(nz, lane >= (c * LANES - off))
            cnt = cnt + plsc.all_reduce_population_count(nz)
        inv = 1.0 / jnp.maximum(cnt.astype(jnp.float32), 1.0)

        def inner(i, accs):
            return tuple(accs[d] + rows_v[j, i, pl.ds(d * LANES, LANES)]
                         for d in range(D_CH))

        zeros = tuple(jnp.zeros((LANES,), jnp.float32) for _ in range(D_CH))
        accs = lax.fori_loop(0, SEQ, inner, zeros, unroll=8)
        for d in range(D_CH):
            out_v[row, pl.ds(d * LANES, LANES)] = accs[d] * inv

    for j in range(PIPE):
        issue(j, j)

    def block_body(k, carry):
        for j in range(PIPE):
            row = k * PIPE + j
            drain(j)
            accumulate(row, j)

            @pl.when(row + PIPE < ROWS_PER_W)
            def _():
                issue(row + PIPE, j)
        return carry

    lax.fori_loop(0, ROWS_PER_W // PIPE, block_body, 0)
    pltpu.sync_copy(out_v, out_hbm.at[pl.ds(base, ROWS_PER_W)])


def kernel(sentence, weight):
    idx = sentence.astype(jnp.int32)

    transpose = pl.kernel(
        _transpose_body,
        out_type=jax.ShapeDtypeStruct((VOCAB * EMBED,), jnp.float32),
        mesh=plsc.VectorSubcoreMesh(core_axis_name="c", subcore_axis_name="s"),
        scratch_types=[
            pltpu.VMEM((EMBED, VCHUNK), jnp.float32),
            pltpu.VMEM((EMBED, VCHUNK), jnp.float32),
            pltpu.VMEM((VCHUNK * EMBED,), jnp.float32),
            pltpu.VMEM((VCHUNK * EMBED,), jnp.float32),
            pltpu.VMEM((V_TAIL, EMBED), jnp.float32),
            pltpu.VMEM((V_TAIL * EMBED,), jnp.float32),
            pltpu.SemaphoreType.DMA,
            pltpu.SemaphoreType.DMA,
            pltpu.SemaphoreType.DMA,
            pltpu.SemaphoreType.DMA,
        ],
        compiler_params=pltpu.CompilerParams(use_tc_tiling_on_sc=True,
                                             needs_layout_passes=False),
    )
    w_lin = transpose(weight.T, weight[N_VCHUNKS * VCHUNK:])
    w_lin = w_lin.reshape(VOCAB, EMBED)

    gather = pl.kernel(
        _gather_body,
        out_type=jax.ShapeDtypeStruct((BATCH, EMBED), jnp.float32),
        mesh=plsc.VectorSubcoreMesh(core_axis_name="c", subcore_axis_name="s"),
        scratch_types=[
            pltpu.VMEM((ROWS_PER_W, SEQ), jnp.int32),
            pltpu.VMEM((PIPE, SEQ, EMBED), jnp.float32),
            pltpu.VMEM((ROWS_PER_W, EMBED), jnp.float32),
        ] + [pltpu.SemaphoreType.DMA] * PIPE,
        compiler_params=pltpu.CompilerParams(use_tc_tiling_on_sc=False,
                                             needs_layout_passes=False),
    )
    return gather(idx, w_lin)


# speedup vs baseline: 2.3527x; 1.0684x over previous
import jax
import jax.numpy as jnp
from jax import lax
from jax.experimental import pallas as pl
from jax.experimental.pallas import tpu as pltpu
from jax.experimental.pallas import tpu_sc as plsc

VOCAB = 1000000
BATCH = 4096
SEQ = 200
CH0 = 104
CH1 = SEQ - CH0
EMBED = 64
NUM_WORKERS = 32
ROWS_PER_W = BATCH // NUM_WORKERS
LANES = 16
D_CH = EMBED // LANES
PIPE = 4
N_CNT = SEQ // LANES + 1

VCHUNK = 128
N_VCHUNKS = VOCAB // VCHUNK
V_TAIL = VOCAB - N_VCHUNKS * VCHUNK
STEPS = (N_VCHUNKS + 2 * NUM_WORKERS - 1) // (2 * NUM_WORKERS)


def _transpose_body(wt_hbm, tail_hbm, out_hbm, buf_a, buf_b, flat_a, flat_b,
                    tail_buf, tail_flat, in_a, in_b, out_a, out_b):
    wid = lax.axis_index("s") * 2 + lax.axis_index("c")
    iota = lax.iota(jnp.int32, LANES)
    d_base = [iota + db * LANES for db in range(EMBED // LANES)]

    def issue_in(cid, buf, sem):
        @pl.when(cid < N_VCHUNKS)
        def _():
            pltpu.async_copy(
                wt_hbm.at[pl.ds(0, EMBED), pl.ds(cid * VCHUNK, VCHUNK)],
                buf, sem)

    def transpose(buf, flat):
        def krot(k, carry):
            rot = jnp.bitwise_and(iota + k, LANES - 1)
            wk = rot * EMBED + iota
            for vb in range(VCHUNK // LANES):
                ridx = rot + vb * LANES
                wbase = wk + vb * (LANES * EMBED)
                for db in range(EMBED // LANES):
                    x = plsc.load_gather(buf, [d_base[db], ridx])
                    plsc.store_scatter(flat, [wbase + db * LANES], x)
            return carry

        lax.fori_loop(0, LANES, krot, 0, unroll=4)

    def substep(j, cid, buf, flat, in_sem, out_sem):
        @pl.when(cid < N_VCHUNKS)
        def _():
            pltpu.make_async_copy(
                wt_hbm.at[pl.ds(0, EMBED), pl.ds(0, VCHUNK)], buf,
                in_sem).wait()

            @pl.when(j > 0)
            def _():
                pltpu.make_async_copy(
                    flat, out_hbm.at[pl.ds(0, VCHUNK * EMBED)],
                    out_sem).wait()

            transpose(buf, flat)
            pltpu.async_copy(
                flat, out_hbm.at[pl.ds(cid * VCHUNK * EMBED, VCHUNK * EMBED)],
                out_sem)
            issue_in(cid + 2 * NUM_WORKERS, buf, in_sem)

    issue_in(wid, buf_a, in_a)
    issue_in(NUM_WORKERS + wid, buf_b, in_b)

    def step(j, carry):
        substep(j, (2 * j) * NUM_WORKERS + wid, buf_a, flat_a, in_a, out_a)
        substep(j, (2 * j + 1) * NUM_WORKERS + wid, buf_b, flat_b, in_b, out_b)
        return carry

    lax.fori_loop(0, STEPS, step, 0)
    pltpu.make_async_copy(flat_a, out_hbm.at[pl.ds(0, VCHUNK * EMBED)],
                          out_a).wait()
    pltpu.make_async_copy(flat_b, out_hbm.at[pl.ds(0, VCHUNK * EMBED)],
                          out_b).wait()

    @pl.when(wid == 0)
    def _():
        pltpu.sync_copy(tail_hbm, tail_buf)

        def body(v, carry):
            for d in range(D_CH):
                tail_flat[pl.ds(v * EMBED + d * LANES, LANES)] = (
                    tail_buf[v, pl.ds(d * LANES, LANES)])
            return carry

        lax.fori_loop(0, V_TAIL, body, 0, unroll=4)
        pltpu.sync_copy(
            tail_flat,
            out_hbm.at[pl.ds(N_VCHUNKS * VCHUNK * EMBED, V_TAIL * EMBED)])


def _gather_body(idx_hbm, w_hbm, out_hbm, idx_v, rows_v, out_v, *sems):
    wid = lax.axis_index("s") * 2 + lax.axis_index("c")
    base = wid * ROWS_PER_W
    pltpu.sync_copy(idx_hbm.at[pl.ds(base, ROWS_PER_W)], idx_v)

    def issue(row, j):
        pltpu.async_copy(w_hbm.at[idx_v.at[row, pl.ds(0, CH0)]],
                         rows_v.at[j, pl.ds(0, CH0)], sems[j])
        pltpu.async_copy(w_hbm.at[idx_v.at[row, pl.ds(CH0, CH1)]],
                         rows_v.at[j, pl.ds(CH0, CH1)], sems[j])

    def drain(j):
        pltpu.make_async_copy(w_hbm.at[pl.ds(0, SEQ)],
                              rows_v.at[j], sems[j]).wait()

    lane = lax.iota(jnp.int32, LANES)

    def accumulate(row, j):
        cnt = jnp.zeros((LANES,), jnp.int32)
        for c in range(N_CNT):
            off = min(c * LANES, SEQ - LANES)
            iv = idx_v[row, pl.ds(off, LANES)]
            nz = iv != 0
            if c * LANES > off:
                nz = jnp.logical_and(nz, lane >= (c * LANES - off))
            cnt = cnt + plsc.all_reduce_population_count(nz)
        inv = 1.0 / jnp.maximum(cnt.astype(jnp.float32), 1.0)

        def inner(i, accs):
            return tuple(accs[d] + rows_v[j, i, pl.ds(d * LANES, LANES)]
                         for d in range(D_CH))

        zeros = tuple(jnp.zeros((LANES,), jnp.float32) for _ in range(D_CH))
        accs = lax.fori_loop(0, SEQ, inner, zeros, unroll=8)
        for d in range(D_CH):
            out_v[row, pl.ds(d * LANES, LANES)] = accs[d] * inv

    for j in range(PIPE):
        issue(j, j)

    def block_body(k, carry):
        for j in range(PIPE):
            row = k * PIPE + j
            drain(j)
            accumulate(row, j)

            @pl.when(row + PIPE < ROWS_PER_W)
            def _():
                issue(row + PIPE, j)
        return carry

    lax.fori_loop(0, ROWS_PER_W // PIPE, block_body, 0)
    pltpu.sync_copy(out_v, out_hbm.at[pl.ds(base, ROWS_PER_W)])


def kernel(sentence, weight):
    idx = sentence.astype(jnp.int32)

    transpose = pl.kernel(
        _transpose_body,
        out_type=jax.ShapeDtypeStruct((VOCAB * EMBED,), jnp.float32),
        mesh=plsc.VectorSubcoreMesh(core_axis_name="c", subcore_axis_name="s"),
        scratch_types=[
            pltpu.VMEM((EMBED, VCHUNK), jnp.float32),
            pltpu.VMEM((EMBED, VCHUNK), jnp.float32),
            pltpu.VMEM((VCHUNK * EMBED,), jnp.float32),
            pltpu.VMEM((VCHUNK * EMBED,), jnp.float32),
            pltpu.VMEM((V_TAIL, EMBED), jnp.float32),
            pltpu.VMEM((V_TAIL * EMBED,), jnp.float32),
            pltpu.SemaphoreType.DMA,
            pltpu.SemaphoreType.DMA,
            pltpu.SemaphoreType.DMA,
            pltpu.SemaphoreType.DMA,
        ],
        compiler_params=pltpu.CompilerParams(use_tc_tiling_on_sc=True,
                                             needs_layout_passes=False),
    )
    w_lin = transpose(weight.T, weight[N_VCHUNKS * VCHUNK:])
    w_lin = w_lin.reshape(VOCAB, EMBED)

    gather = pl.kernel(
        _gather_body,
        out_type=jax.ShapeDtypeStruct((BATCH, EMBED), jnp.float32),
        mesh=plsc.VectorSubcoreMesh(core_axis_name="c", subcore_axis_name="s"),
        scratch_types=[
            pltpu.VMEM((ROWS_PER_W, SEQ), jnp.int32),
            pltpu.VMEM((PIPE, SEQ, EMBED), jnp.float32),
            pltpu.VMEM((ROWS_PER_W, EMBED), jnp.float32),
        ] + [pltpu.SemaphoreType.DMA] * PIPE,
        compiler_params=pltpu.CompilerParams(use_tc_tiling_on_sc=False,
                                             needs_layout_passes=False),
    )
    return gather(idx, w_lin)
